# baseline (speedup 1.0000x reference)
import jax
import jax.numpy as jnp
from jax import lax
from jax.experimental import pallas as pl
from jax.experimental.pallas import tpu as pltpu
from jax.experimental.pallas import tpu_sc as plsc

N = 10000
E = 320000
D = 128
DH = D // 2
NEG_SLOPE = 0.2

NC = 2
NS = 16
EDGES_PER_T = E // NS
CH = 128
NCHT = 157
EDGES_PAD = NCHT * CH
NPAIR = (NCHT - 1) // 2
RTILE = 640
NACC = NS * RTILE


def _prep_body(x_ref, w_ref, a_ref, h_ref, s1_ref, s2_ref):
    h = jnp.dot(x_ref[...], w_ref[...], preferred_element_type=jnp.float32)
    h_ref[...] = h
    s1_ref[...] = jnp.dot(h, a_ref[0, :D], preferred_element_type=jnp.float32)
    s2_ref[...] = jnp.dot(h, a_ref[0, D:], preferred_element_type=jnp.float32)


def _sc_body(h2, s1, s2, srcs, dsts, out,
             acc, acc1, src_v, dst_v, dstg_v, w_v, s1g0, s1g1, s2g0, s2g1,
             rows0, rows1, zb1, rs_v, ri_v,
             sem_r0, sem_r1, sem_10, sem_11, sem_20, sem_21):
    cid = lax.axis_index("c")
    sid = lax.axis_index("s")

    s1g = (s1g0, s1g1)
    s2g = (s2g0, s2g1)
    rows = (rows0, rows1)
    sem_r = (sem_r0, sem_r1)
    sem_1 = (sem_10, sem_11)
    sem_2 = (sem_20, sem_21)

    zv = jnp.zeros((16,), jnp.float32)

    def zrow(r, carry):
        for j in range(DH // 16):
            rows0[r, pl.ds(j * 16, 16)] = zv
        return carry

    lax.fori_loop(0, CH, zrow, 0)
    for i in range(RTILE // 16):
        zb1[pl.ds(i * 16, 16)] = zv
    rbase0 = sid * RTILE
    for k in range(RTILE // CH):
        pltpu.sync_copy(rows0, acc.at[pl.ds(rbase0 + k * CH, CH)])
    pltpu.sync_copy(zb1, acc1.at[pl.ds(rbase0, RTILE)])

    pltpu.sync_copy(srcs.at[sid], src_v)
    pltpu.sync_copy(dsts.at[sid], dst_v)

    def trow(r, carry):
        for j in range(CH // 16):
            d = dst_v[r, pl.ds(j * 16, 16)]
            dstg_v[r, pl.ds(j * 16, 16)] = d + d + cid
        return carry

    lax.fori_loop(0, NCHT, trow, 0)

    plsc.subcore_barrier()

    def start_gathers(c, b):
        pltpu.async_copy(h2.at[dstg_v.at[c]], rows[b], sem_r[b])
        pltpu.async_copy(s1.at[src_v.at[c]], s1g[b], sem_1[b])
        pltpu.async_copy(s2.at[dst_v.at[c]], s2g[b], sem_2[b])

    def compute_chunk(c, b, mask_last=False):
        pltpu.make_async_copy(s1.at[src_v.at[c]], s1g[b], sem_1[b]).wait()
        pltpu.make_async_copy(s2.at[dst_v.at[c]], s2g[b], sem_2[b]).wait()
        for i in range(CH // 16):
            if mask_last and i >= 2:
                w_v[pl.ds(i * 16, 16)] = zv
            else:
                logit = s1g[b][pl.ds(i * 16, 16)] + s2g[b][pl.ds(i * 16, 16)]
                w = jnp.exp(jnp.where(logit > 0.0, -logit, (-NEG_SLOPE) * logit))
                w_v[pl.ds(i * 16, 16)] = w

        pltpu.make_async_copy(h2.at[dstg_v.at[c]], rows[b], sem_r[b]).wait()

        @plsc.parallel_loop(0, CH, unroll=4)
        def scale(e):
            wv = plsc.load_gather(w_v, [jnp.broadcast_to(e, (16,)).astype(jnp.int32)])
            for j in range(DH // 16):
                rows[b][e, pl.ds(j * 16, 16)] = rows[b][e, pl.ds(j * 16, 16)] * wv

        pltpu.sync_copy(w_v, acc1.at[src_v.at[c]], add=True)
        pltpu.sync_copy(rows[b], acc.at[src_v.at[c]], add=True)

    start_gathers(0, 0)

    def pair_body(c0, carry):
        c = 2 * c0
        start_gathers(c + 1, 1)
        compute_chunk(c, 0)
        start_gathers(c + 2, 0)
        compute_chunk(c + 1, 1)
        return carry

    lax.fori_loop(0, NPAIR, pair_body, 0)
    compute_chunk(NCHT - 1, 0, mask_last=True)

    plsc.subcore_barrier()

    def finish_chunk(rbase, nrows):
        pltpu.sync_copy(acc.at[pl.ds(rbase, nrows)], rows0.at[pl.ds(0, nrows)])
        pltpu.sync_copy(acc1.at[pl.ds(rbase, nrows)], rs_v.at[pl.ds(0, nrows)])
        for g in range(nrows // 16):
            ri_v[pl.ds(g * 16, 16)] = 1.0 / rs_v[pl.ds(g * 16, 16)]

        @plsc.parallel_loop(0, nrows, unroll=4)
        def fin(e):
            ri = plsc.load_gather(ri_v, [jnp.broadcast_to(e, (16,)).astype(jnp.int32)])
            for j in range(DH // 16):
                v = rows0[e, pl.ds(j * 16, 16)] * ri
                rows0[e, pl.ds(j * 16, 16)] = jnp.where(v > 0.0, v,
                                                        jnp.exp(v) - 1.0)

        pltpu.sync_copy(rows0.at[pl.ds(0, nrows)],
                        out.at[pl.ds(rbase, nrows), pl.ds(cid * DH, DH)])

    for k in range(RTILE // CH):
        if k < 3:
            finish_chunk(rbase0 + k * CH, CH)
        else:
            @pl.when(sid < NS - 1)
            def _():
                finish_chunk(rbase0 + k * CH, CH)

    @pl.when(sid == NS - 1)
    def _():
        finish_chunk((NS - 1) * RTILE + 3 * CH, 16)


_sc_call = pl.kernel(
    _sc_body,
    out_type=jax.ShapeDtypeStruct((N, D), jnp.float32),
    mesh=plsc.VectorSubcoreMesh(core_axis_name="c", subcore_axis_name="s",
                                num_cores=NC, num_subcores=NS),
    compiler_params=pltpu.CompilerParams(use_tc_tiling_on_sc=False,
                                         needs_layout_passes=False),
    scratch_types=[
        pltpu.VMEM_SHARED((NACC, DH), jnp.float32),
        pltpu.VMEM_SHARED((NACC,), jnp.float32),
        pltpu.VMEM((NCHT, CH), jnp.int32),
        pltpu.VMEM((NCHT, CH), jnp.int32),
        pltpu.VMEM((NCHT, CH), jnp.int32),
        pltpu.VMEM((CH,), jnp.float32),
        pltpu.VMEM((CH,), jnp.float32),
        pltpu.VMEM((CH,), jnp.float32),
        pltpu.VMEM((CH,), jnp.float32),
        pltpu.VMEM((CH,), jnp.float32),
        pltpu.VMEM((CH, DH), jnp.float32),
        pltpu.VMEM((CH, DH), jnp.float32),
        pltpu.VMEM((RTILE,), jnp.float32),
        pltpu.VMEM((CH,), jnp.float32),
        pltpu.VMEM((CH,), jnp.float32),
        pltpu.SemaphoreType.DMA,
        pltpu.SemaphoreType.DMA,
        pltpu.SemaphoreType.DMA,
        pltpu.SemaphoreType.DMA,
        pltpu.SemaphoreType.DMA,
        pltpu.SemaphoreType.DMA,
    ],
)


def kernel(x, edge_index, W, a):
    ei = edge_index.astype(jnp.int32)
    pad = ((0, 0), (0, EDGES_PAD - EDGES_PER_T))
    srcs = jnp.pad(ei[0].reshape(NS, EDGES_PER_T), pad).reshape(NS, NCHT, CH)
    dsts = jnp.pad(ei[1].reshape(NS, EDGES_PER_T), pad).reshape(NS, NCHT, CH)
    h, s1, s2 = pl.pallas_call(
        _prep_body,
        out_shape=(jax.ShapeDtypeStruct((N, D), jnp.float32),
                   jax.ShapeDtypeStruct((N,), jnp.float32),
                   jax.ShapeDtypeStruct((N,), jnp.float32)),
    )(x, W, a)
    h2 = h.reshape(2 * N, DH)
    return _sc_call(h2, s1, s2, srcs, dsts)


# speedup vs baseline: 1.1266x; 1.1266x over previous
import jax
import jax.numpy as jnp
from jax import lax
from jax.experimental import pallas as pl
from jax.experimental.pallas import tpu as pltpu
from jax.experimental.pallas import tpu_sc as plsc

N = 10000
E = 320000
D = 128
DAUG = 144
NEG_SLOPE = 0.2

NC = 2
NS = 16
NW = NC * NS
EDGES_PER_W = E // NW
CHUNK = 100
CP = 112
NCHUNK = EDGES_PER_W // CHUNK
NPAIR = NCHUNK // 2 - 1
ROWS_PER_TILE = N // NS


def _prep_body(x_ref, w_ref, a_ref, a2_ref, haug_ref, s1_ref):
    h = jnp.dot(x_ref[...], w_ref[...], preferred_element_type=jnp.float32)
    haug_ref[:, :D] = h
    haug_ref[:, D:] = jnp.dot(h, a2_ref[...], preferred_element_type=jnp.float32)
    s1_ref[...] = jnp.dot(h, a_ref[0, :D], preferred_element_type=jnp.float32)


def _sc_body(haug, s1, srcs, dsts, part,
             acc, s1_v, srcc0, srcc1, srcc2, srcc3, dstc0, dstc1, dstc2, dstc3,
             w_v, rows0, rows1,
             sem_i0, sem_i1, sem_i2, sem_i3, sem_r0, sem_r1):
    cid = lax.axis_index("c")
    sid = lax.axis_index("s")
    wid = cid * NS + sid

    srcc = (srcc0, srcc1, srcc2, srcc3)
    dstc = (dstc0, dstc1, dstc2, dstc3)
    rows = (rows0, rows1)
    sem_i = (sem_i0, sem_i1, sem_i2, sem_i3)
    sem_r = (sem_r0, sem_r1)

    zv = jnp.zeros((16,), jnp.float32)
    ziv = jnp.zeros((16,), jnp.int32)

    for b in range(4):
        srcc[b][pl.ds(CHUNK - 4, 16)] = ziv
        dstc[b][pl.ds(CHUNK - 4, 16)] = ziv

    def zrow(r, carry):
        for j in range(DAUG // 16):
            rows0[r, pl.ds(j * 16, 16)] = zv
        return carry

    lax.fori_loop(0, CHUNK, zrow, 0)
    base = sid * ROWS_PER_TILE
    for k in range(ROWS_PER_TILE // CHUNK):
        pltpu.sync_copy(rows0, acc.at[pl.ds(base + k * CHUNK, CHUNK)])
    rem = ROWS_PER_TILE % CHUNK
    if rem:
        pltpu.sync_copy(rows0.at[pl.ds(0, rem)],
                        acc.at[pl.ds(base + (ROWS_PER_TILE // CHUNK) * CHUNK, rem)])

    pltpu.sync_copy(s1, s1_v)

    plsc.subcore_barrier()

    def start_idx(c, i):
        pltpu.async_copy(srcs.at[wid, c], srcc[i].at[pl.ds(0, CHUNK)], sem_i[i])
        pltpu.async_copy(dsts.at[wid, c], dstc[i].at[pl.ds(0, CHUNK)], sem_i[i])

    def wait_idx(i):
        pltpu.make_async_copy(srcs.at[wid, 0], srcc[i].at[pl.ds(0, CHUNK)],
                              sem_i[i]).wait()
        pltpu.make_async_copy(dsts.at[wid, 0], dstc[i].at[pl.ds(0, CHUNK)],
                              sem_i[i]).wait()

    def start_rows(i, b):
        pltpu.async_copy(haug.at[dstc[i].at[pl.ds(0, CHUNK)]], rows[b], sem_r[b])

    def compute_chunk(i, b):
        pltpu.make_async_copy(haug.at[dstc[i].at[pl.ds(0, CHUNK)]], rows[b],
                              sem_r[b]).wait()

        for g in range(CP // 16):
            src16 = srcc[i][pl.ds(g * 16, 16)]
            e16 = jnp.minimum(
                lax.iota(jnp.int32, 16) + jnp.int32(g * 16), jnp.int32(CHUNK - 1))
            s1v = plsc.load_gather(s1_v, [src16])
            s2v = plsc.load_gather(rows[b], [e16, jnp.full((16,), D, jnp.int32)])
            logit = s1v + s2v
            w = jnp.exp(jnp.where(logit > 0.0, -logit, (-NEG_SLOPE) * logit))
            w_v[pl.ds(g * 16, 16)] = w

        onehot = jnp.where(lax.iota(jnp.int32, 16) == 0, 1.0, 0.0)

        @plsc.parallel_loop(0, CHUNK, unroll=4)
        def scale(e):
            wv = plsc.load_gather(w_v, [jnp.broadcast_to(e, (16,)).astype(jnp.int32)])
            for j in range(D // 16):
                rows[b][e, pl.ds(j * 16, 16)] = rows[b][e, pl.ds(j * 16, 16)] * wv
            rows[b][e, pl.ds(D, 16)] = wv * onehot

        pltpu.sync_copy(rows[b], acc.at[srcc[i].at[pl.ds(0, CHUNK)]], add=True)

    for i in range(4):
        start_idx(i, i)
    wait_idx(0)
    start_rows(0, 0)

    def quad_body(q, carry):
        c = 4 * q
        for r in range(4):
            wait_idx((r + 1) % 4)
            start_rows((r + 1) % 4, (r + 1) % 2)
            compute_chunk(r, r % 2)
            start_idx(c + r + 4, r)
        return carry

    lax.fori_loop(0, NCHUNK // 4 - 1, quad_body, 0)

    for r in range(4):
        if r < 3:
            wait_idx((r + 1) % 4)
            start_rows((r + 1) % 4, (r + 1) % 2)
        compute_chunk(r, r % 2)

    plsc.subcore_barrier()
    pltpu.sync_copy(acc.at[pl.ds(base, ROWS_PER_TILE)],
                    part.at[cid, pl.ds(base, ROWS_PER_TILE)])


_sc_call = pl.kernel(
    _sc_body,
    out_type=jax.ShapeDtypeStruct((NC, N, DAUG), jnp.float32),
    mesh=plsc.VectorSubcoreMesh(core_axis_name="c", subcore_axis_name="s",
                                num_cores=NC, num_subcores=NS),
    compiler_params=pltpu.CompilerParams(use_tc_tiling_on_sc=False,
                                         needs_layout_passes=False),
    scratch_types=[
        pltpu.VMEM_SHARED((N, DAUG), jnp.float32),
        pltpu.VMEM((N,), jnp.float32),
        pltpu.VMEM((CP,), jnp.int32),
        pltpu.VMEM((CP,), jnp.int32),
        pltpu.VMEM((CP,), jnp.int32),
        pltpu.VMEM((CP,), jnp.int32),
        pltpu.VMEM((CP,), jnp.int32),
        pltpu.VMEM((CP,), jnp.int32),
        pltpu.VMEM((CP,), jnp.int32),
        pltpu.VMEM((CP,), jnp.int32),
        pltpu.VMEM((CP,), jnp.float32),
        pltpu.VMEM((CHUNK, DAUG), jnp.float32),
        pltpu.VMEM((CHUNK, DAUG), jnp.float32),
        pltpu.SemaphoreType.DMA,
        pltpu.SemaphoreType.DMA,
        pltpu.SemaphoreType.DMA,
        pltpu.SemaphoreType.DMA,
        pltpu.SemaphoreType.DMA,
        pltpu.SemaphoreType.DMA,
    ],
)


def _fin_body(p_ref, o_ref):
    p = p_ref[0] + p_ref[1]
    hp = p[:, :D] / p[:, D:D + 1]
    o_ref[...] = jnp.where(hp > 0.0, hp, jnp.exp(hp) - 1.0)


def kernel(x, edge_index, W, a):
    ei = edge_index.astype(jnp.int32)
    src = ei[0].reshape(NW, NCHUNK, CHUNK)
    dst = ei[1].reshape(NW, NCHUNK, CHUNK)
    a2 = jnp.pad(a[0, D:][:, None], ((0, 0), (0, DAUG - D - 1)))
    haug, s1 = pl.pallas_call(
        _prep_body,
        out_shape=(jax.ShapeDtypeStruct((N, DAUG), jnp.float32),
                   jax.ShapeDtypeStruct((N,), jnp.float32)),
    )(x, W, a, a2)
    part = _sc_call(haug, s1, src, dst)
    return pl.pallas_call(
        _fin_body,
        out_shape=jax.ShapeDtypeStruct((N, D), jnp.float32),
    )(part)


# speedup vs baseline: 1.3404x; 1.1898x over previous
import jax
import jax.numpy as jnp
from jax import lax
from jax.experimental import pallas as pl
from jax.experimental.pallas import tpu as pltpu
from jax.experimental.pallas import tpu_sc as plsc

N = 10000
E = 320000
D = 128
NEG_SLOPE = 0.2

NC = 2
NS = 16
NW = NC * NS
EDGES_PER_W = E // NW
CHUNK = 100
CP = 104
NCHUNK = EDGES_PER_W // CHUNK
ROWS_PER_TILE = N // NS
NSUM = 10240


def _prep_body(x_ref, w_ref, a_ref, h_ref, s1_ref, s2_ref):
    h = jnp.dot(x_ref[...], w_ref[...], preferred_element_type=jnp.float32)
    h_ref[...] = h
    s1_ref[...] = jnp.dot(h, a_ref[0, :D], preferred_element_type=jnp.float32)
    s2_ref[...] = jnp.dot(h, a_ref[0, D:], preferred_element_type=jnp.float32)


def _sc_body(h, s1, s2, srcs, dsts, part, psum,
             acc, acc1, s1_v, src_v, dstc0, dstc1, dstc2, dstc3,
             w_v, s2g0, s2g1, rows0, rows1, zb1,
             sem_i0, sem_i1, sem_i2, sem_i3, sem_r0, sem_r1, sem_20, sem_21):
    cid = lax.axis_index("c")
    sid = lax.axis_index("s")
    wid = cid * NS + sid

    dstc = (dstc0, dstc1, dstc2, dstc3)
    s2g = (s2g0, s2g1)
    rows = (rows0, rows1)
    sem_i = (sem_i0, sem_i1, sem_i2, sem_i3)
    sem_r = (sem_r0, sem_r1)
    sem_2 = (sem_20, sem_21)

    zv = jnp.zeros((16,), jnp.float32)

    def zrow(r, carry):
        for j in range(D // 16):
            rows0[r, pl.ds(j * 16, 16)] = zv
        return carry

    lax.fori_loop(0, CHUNK, zrow, 0)
    for i in range(NSUM // NS // 16):
        zb1[pl.ds(i * 16, 16)] = zv
    base = sid * ROWS_PER_TILE
    for k in range(ROWS_PER_TILE // CHUNK):
        pltpu.sync_copy(rows0, acc.at[pl.ds(base + k * CHUNK, CHUNK)])
    rem = ROWS_PER_TILE % CHUNK
    if rem:
        pltpu.sync_copy(rows0.at[pl.ds(0, rem)],
                        acc.at[pl.ds(base + (ROWS_PER_TILE // CHUNK) * CHUNK, rem)])
    pltpu.sync_copy(zb1, acc1.at[pl.ds(sid * (NSUM // NS), NSUM // NS)])

    pltpu.sync_copy(s1, s1_v)
    pltpu.sync_copy(srcs.at[wid], src_v)

    plsc.subcore_barrier()

    def start_idx(c, i):
        pltpu.async_copy(dsts.at[wid, c], dstc[i].at[pl.ds(0, CHUNK)], sem_i[i])

    def wait_idx(i):
        pltpu.make_async_copy(dsts.at[wid, 0], dstc[i].at[pl.ds(0, CHUNK)],
                              sem_i[i]).wait()

    def start_gathers(i, b):
        pltpu.async_copy(h.at[dstc[i].at[pl.ds(0, CHUNK)]], rows[b], sem_r[b])
        pltpu.async_copy(s2.at[dstc[i].at[pl.ds(0, CHUNK)]],
                         s2g[b].at[pl.ds(0, CHUNK)], sem_2[b])

    def compute_chunk(c, i, b):
        pltpu.make_async_copy(s2.at[dstc[i].at[pl.ds(0, CHUNK)]],
                              s2g[b].at[pl.ds(0, CHUNK)], sem_2[b]).wait()
        for g in range(CHUNK // 16 + 1):
            if g < CHUNK // 16:
                off = g * 16
                src16 = src_v[c, pl.ds(off, 16)]
            else:
                off = CHUNK - 12
                col16 = jnp.minimum(lax.iota(jnp.int32, 16) + jnp.int32(off),
                                    jnp.int32(CHUNK - 1))
                src16 = plsc.load_gather(src_v, [jnp.broadcast_to(c, (16,)), col16])
            s1v = plsc.load_gather(s1_v, [src16])
            logit = s1v + s2g[b][pl.ds(off, 16)]
            w = jnp.exp(jnp.where(logit > 0.0, -logit, (-NEG_SLOPE) * logit))
            w_v[pl.ds(off, 16)] = w

        pltpu.make_async_copy(h.at[dstc[i].at[pl.ds(0, CHUNK)]], rows[b],
                              sem_r[b]).wait()

        @plsc.parallel_loop(0, CHUNK, unroll=4)
        def scale(e):
            wv = plsc.load_gather(w_v, [jnp.broadcast_to(e, (16,)).astype(jnp.int32)])
            for j in range(D // 16):
                rows[b][e, pl.ds(j * 16, 16)] = rows[b][e, pl.ds(j * 16, 16)] * wv

        pltpu.sync_copy(w_v.at[pl.ds(0, CHUNK)], acc1.at[src_v.at[c]], add=True)
        pltpu.sync_copy(rows[b], acc.at[src_v.at[c]], add=True)

    for i in range(4):
        start_idx(i, i)
    wait_idx(0)
    start_gathers(0, 0)

    def quad_body(q, carry):
        c = 4 * q
        for r in range(4):
            wait_idx((r + 1) % 4)
            start_gathers((r + 1) % 4, (r + 1) % 2)
            compute_chunk(c + r, r, r % 2)
            start_idx(c + r + 4, r)
        return carry

    lax.fori_loop(0, NCHUNK // 4 - 1, quad_body, 0)

    for r in range(4):
        if r < 3:
            wait_idx((r + 1) % 4)
            start_gathers((r + 1) % 4, (r + 1) % 2)
        compute_chunk(NCHUNK - 4 + r, r, r % 2)

    plsc.subcore_barrier()
    pltpu.sync_copy(acc.at[pl.ds(base, ROWS_PER_TILE)],
                    part.at[cid, pl.ds(base, ROWS_PER_TILE)])
    pltpu.sync_copy(acc1.at[pl.ds(sid * (NSUM // NS), NSUM // NS)],
                    psum.at[cid, pl.ds(sid * (NSUM // NS), NSUM // NS)])


_sc_call = pl.kernel(
    _sc_body,
    out_type=(jax.ShapeDtypeStruct((NC, N, D), jnp.float32),
              jax.ShapeDtypeStruct((NC, NSUM), jnp.float32)),
    mesh=plsc.VectorSubcoreMesh(core_axis_name="c", subcore_axis_name="s",
                                num_cores=NC, num_subcores=NS),
    compiler_params=pltpu.CompilerParams(use_tc_tiling_on_sc=False,
                                         needs_layout_passes=False),
    scratch_types=[
        pltpu.VMEM_SHARED((N, D), jnp.float32),
        pltpu.VMEM_SHARED((NSUM,), jnp.float32),
        pltpu.VMEM((N,), jnp.float32),
        pltpu.VMEM((NCHUNK, CHUNK), jnp.int32),
        pltpu.VMEM((CP,), jnp.int32),
        pltpu.VMEM((CP,), jnp.int32),
        pltpu.VMEM((CP,), jnp.int32),
        pltpu.VMEM((CP,), jnp.int32),
        pltpu.VMEM((CP,), jnp.float32),
        pltpu.VMEM((CP,), jnp.float32),
        pltpu.VMEM((CP,), jnp.float32),
        pltpu.VMEM((CHUNK, D), jnp.float32),
        pltpu.VMEM((CHUNK, D), jnp.float32),
        pltpu.VMEM((NSUM // NS,), jnp.float32),
        pltpu.SemaphoreType.DMA,
        pltpu.SemaphoreType.DMA,
        pltpu.SemaphoreType.DMA,
        pltpu.SemaphoreType.DMA,
        pltpu.SemaphoreType.DMA,
        pltpu.SemaphoreType.DMA,
        pltpu.SemaphoreType.DMA,
        pltpu.SemaphoreType.DMA,
    ],
)


def _fin_body(p_ref, r_ref, o_ref):
    p = p_ref[0] + p_ref[1]
    r = r_ref[0, :N] + r_ref[1, :N]
    hp = p / jnp.reshape(r, (N, 1))
    o_ref[...] = jnp.where(hp > 0.0, hp, jnp.exp(hp) - 1.0)


def kernel(x, edge_index, W, a):
    ei = edge_index.astype(jnp.int32)
    src = ei[0].reshape(NW, NCHUNK, CHUNK)
    dst = ei[1].reshape(NW, NCHUNK, CHUNK)
    h, s1, s2 = pl.pallas_call(
        _prep_body,
        out_shape=(jax.ShapeDtypeStruct((N, D), jnp.float32),
                   jax.ShapeDtypeStruct((N,), jnp.float32),
                   jax.ShapeDtypeStruct((N,), jnp.float32)),
    )(x, W, a)
    part, psum = _sc_call(h, s1, s2, src, dst)
    return pl.pallas_call(
        _fin_body,
        out_shape=jax.ShapeDtypeStruct((N, D), jnp.float32),
    )(part, psum)
